# Initial kernel scaffold; baseline (speedup 1.0000x reference)
import jax
import jax.numpy as jnp
from jax.experimental import pallas as pl


def kernel(h_tag, h_vid, pos_tag_idx, pos_vid_idx, neg_tag_idx, neg_vid_idx):
    raise NotImplementedError("write your pallas kernel here")


# speedup vs baseline: 2.7612x; 2.7612x over previous
import functools

import jax
import jax.numpy as jnp
from jax import lax
from jax.experimental import pallas as pl
from jax.experimental.pallas import tpu as pltpu
from jax.experimental.pallas import tpu_sc as plsc

N_TAG = 1000
N_VID = 16384
D = 768
E = 50000
FLAT = N_TAG * N_VID

NC = 2
NS = 16
L = 16

CH = 3136
NVREG = CH // L
ROWS_PER_SC = N_TAG // NC
R = 25
NPASS = ROWS_PER_SC // R
SLICE = R * N_VID
SHARE = SLICE // NS
SINK_PAD = 128
ZCH = 6400
HUGE = 0x7FFFFFF0

_mesh = plsc.VectorSubcoreMesh(
    core_axis_name="c", subcore_axis_name="s", num_cores=NC, num_subcores=NS
)


@functools.partial(
    pl.kernel,
    out_type=(
        jax.ShapeDtypeStruct((FLAT,), jnp.float32),
        jax.ShapeDtypeStruct((FLAT,), jnp.float32),
    ),
    mesh=_mesh,
    scratch_types=[
        pltpu.VMEM((CH,), jnp.int32),
        pltpu.VMEM((CH,), jnp.int32),
        pltpu.VMEM((CH,), jnp.int32),
        pltpu.VMEM((CH,), jnp.int32),
        pltpu.VMEM((CH,), jnp.int32),
        pltpu.VMEM((CH,), jnp.int32),
        pltpu.VMEM((CH,), jnp.float32),
        pltpu.VMEM((ZCH,), jnp.float32),
        pltpu.VMEM_SHARED((SLICE + SINK_PAD,), jnp.float32),
        pltpu.VMEM_SHARED((SLICE + SINK_PAD,), jnp.float32),
    ],
)
def _sc_counts(pt, pv, nt, nv, out_all, out_pos,
               tbuf, vbuf, lpos, lneg, idxp, idxn, ones, zeros,
               acc_all, acc_pos):
    c = lax.axis_index("c")
    s = lax.axis_index("s")
    base = jnp.minimum(s * CH, E - CH)
    lane = lax.iota(jnp.int32, L)

    def build(t_hbm, v_hbm, lbuf):
        pltpu.sync_copy(t_hbm.at[pl.ds(base, CH)], tbuf)
        pltpu.sync_copy(v_hbm.at[pl.ds(base, CH)], vbuf)

        def body(j, _):
            t = tbuf[pl.ds(j * L, L)]
            v = vbuf[pl.ds(j * L, L)]
            flat = t * N_VID + v
            gpos = base + j * L + lane
            mine = gpos >= s * CH
            lbuf[pl.ds(j * L, L)] = jnp.where(mine, flat, HUGE)
            return 0

        lax.fori_loop(0, NVREG, body, 0)

    build(pt, pv, lpos)
    build(nt, nv, lneg)

    def fill(j, _):
        ones[pl.ds(j * L, L)] = jnp.full((L,), 1.0, jnp.float32)
        return 0

    lax.fori_loop(0, NVREG, fill, 0)

    def fillz(j, _):
        zeros[pl.ds(j * L, L)] = jnp.zeros((L,), jnp.float32)
        return 0

    lax.fori_loop(0, ZCH // L, fillz, 0)

    def zchunk(k, _):
        off = s * SHARE + k * ZCH
        pltpu.sync_copy(zeros, acc_all.at[pl.ds(off, ZCH)])
        pltpu.sync_copy(zeros, acc_pos.at[pl.ds(off, ZCH)])
        return 0

    lax.fori_loop(0, SHARE // ZCH, zchunk, 0)
    plsc.subcore_barrier()

    def do_pass(p, _):
        lo = (c * ROWS_PER_SC + p * R) * N_VID

        def mk_idx(lbuf, ibuf):
            def body(j, _):
                u = lbuf[pl.ds(j * L, L)] - lo
                inr = (u >= 0) & (u < SLICE)
                sink = SLICE + ((j & 7) * L) + lane
                ibuf[pl.ds(j * L, L)] = jnp.where(inr, u, sink)
                return 0

            lax.fori_loop(0, NVREG, body, 0)

        mk_idx(lpos, idxp)
        mk_idx(lneg, idxn)

        pltpu.sync_copy(ones, acc_all.at[idxp], add=True)
        pltpu.sync_copy(ones, acc_pos.at[idxp], add=True)
        pltpu.sync_copy(ones, acc_all.at[idxn], add=True)
        plsc.subcore_barrier()

        off = s * SHARE
        pltpu.sync_copy(acc_all.at[pl.ds(off, SHARE)],
                        out_all.at[pl.ds(lo + off, SHARE)])
        pltpu.sync_copy(acc_pos.at[pl.ds(off, SHARE)],
                        out_pos.at[pl.ds(lo + off, SHARE)])
        plsc.subcore_barrier()

        z = zeros.at[pl.ds(0, CH)]
        pltpu.sync_copy(z, acc_all.at[idxp])
        pltpu.sync_copy(z, acc_pos.at[idxp])
        pltpu.sync_copy(z, acc_all.at[idxn])
        plsc.subcore_barrier()
        return 0

    lax.fori_loop(0, NPASS, do_pass, 0)


BT = 200
BV = 2048


def _tc_body(a_ref, b_ref, cnt_ref, o_ref):
    s = lax.dot_general(
        a_ref[...], b_ref[...],
        dimension_numbers=(((1,), (1,)), ((), ())),
        preferred_element_type=jnp.float32,
    )
    o_ref[...] = s * cnt_ref[...]


def _tc_score(h_tag, h_vid, counts):
    return pl.pallas_call(
        _tc_body,
        grid=(N_TAG // BT, N_VID // BV),
        in_specs=[
            pl.BlockSpec((BT, D), lambda i, j: (i, 0)),
            pl.BlockSpec((BV, D), lambda i, j: (j, 0)),
            pl.BlockSpec((BT, BV), lambda i, j: (i, j)),
        ],
        out_specs=pl.BlockSpec((BT, BV), lambda i, j: (i, j)),
        out_shape=jax.ShapeDtypeStruct((N_TAG, N_VID), jnp.float32),
    )(h_tag, h_vid, counts)


def kernel(h_tag, h_vid, pos_tag_idx, pos_vid_idx, neg_tag_idx, neg_vid_idx):
    counts_all, counts_pos = _sc_counts(
        pos_tag_idx, pos_vid_idx, neg_tag_idx, neg_vid_idx
    )
    counts_all = counts_all.reshape(N_TAG, N_VID)
    labels = counts_pos.reshape(N_TAG, N_VID)
    cls_score = _tc_score(h_tag, h_vid, counts_all)
    return (cls_score, labels)


# speedup vs baseline: 3.1469x; 1.1397x over previous
import functools

import jax
import jax.numpy as jnp
from jax import lax
from jax.experimental import pallas as pl
from jax.experimental.pallas import tpu as pltpu
from jax.experimental.pallas import tpu_sc as plsc

N_TAG = 1000
N_VID = 16384
D = 768
E = 50000
FLAT = N_TAG * N_VID

NC = 2
NS = 16
L = 16

CH = 3136
NVREG = CH // L
ROWS_PER_SC = N_TAG // NC
R = 50
NPASS = ROWS_PER_SC // R
SLICE = R * N_VID
SHARE = SLICE // NS
SINK_PAD = 128
ZCH = 6400
HUGE = 0x7FFFFFF0

_mesh = plsc.VectorSubcoreMesh(
    core_axis_name="c", subcore_axis_name="s", num_cores=NC, num_subcores=NS
)


@functools.partial(
    pl.kernel,
    out_type=(
        jax.ShapeDtypeStruct((FLAT,), jnp.float32),
        jax.ShapeDtypeStruct((FLAT,), jnp.float32),
    ),
    mesh=_mesh,
    scratch_types=[
        pltpu.VMEM((CH,), jnp.int32),
        pltpu.VMEM((CH,), jnp.int32),
        pltpu.VMEM((CH,), jnp.int32),
        pltpu.VMEM((CH,), jnp.int32),
        pltpu.VMEM((CH,), jnp.int32),
        pltpu.VMEM((CH,), jnp.int32),
        pltpu.VMEM((CH,), jnp.float32),
        pltpu.VMEM((ZCH,), jnp.float32),
        pltpu.VMEM_SHARED((SLICE + SINK_PAD,), jnp.float32),
    ],
)
def _sc_counts(pt, pv, nt, nv, out_all, out_pos,
               tbuf, vbuf, lpos, lneg, idxp, idxn, ones, zeros,
               acc):
    c = lax.axis_index("c")
    s = lax.axis_index("s")
    base = jnp.minimum(s * CH, E - CH)
    lane = lax.iota(jnp.int32, L)

    def build(t_hbm, v_hbm, lbuf):
        pltpu.sync_copy(t_hbm.at[pl.ds(base, CH)], tbuf)
        pltpu.sync_copy(v_hbm.at[pl.ds(base, CH)], vbuf)

        def body(j, _):
            t = tbuf[pl.ds(j * L, L)]
            v = vbuf[pl.ds(j * L, L)]
            flat = t * N_VID + v
            gpos = base + j * L + lane
            mine = gpos >= s * CH
            lbuf[pl.ds(j * L, L)] = jnp.where(mine, flat, HUGE)
            return 0

        lax.fori_loop(0, NVREG, body, 0)

    build(pt, pv, lpos)
    build(nt, nv, lneg)

    def fill(j, _):
        ones[pl.ds(j * L, L)] = jnp.full((L,), 1.0, jnp.float32)
        return 0

    lax.fori_loop(0, NVREG, fill, 0)

    def fillz(j, _):
        zeros[pl.ds(j * L, L)] = jnp.zeros((L,), jnp.float32)
        return 0

    lax.fori_loop(0, ZCH // L, fillz, 0)

    def zchunk(k, _):
        off = s * SHARE + k * ZCH
        pltpu.sync_copy(zeros, acc.at[pl.ds(off, ZCH)])
        return 0

    lax.fori_loop(0, SHARE // ZCH, zchunk, 0)
    plsc.subcore_barrier()

    def mk_idx(lbuf, ibuf, lo):
        def body(j, _):
            u = lbuf[pl.ds(j * L, L)] - lo
            inr = (u >= 0) & (u < SLICE)
            sink = SLICE + ((j & 7) * L) + lane
            ibuf[pl.ds(j * L, L)] = jnp.where(inr, u, sink)
            return 0

        lax.fori_loop(0, NVREG, body, 0)

    z = zeros.at[pl.ds(0, CH)]

    def phase(lbufs, ibufs, out):
        def do_pass(p, _):
            lo = (c * ROWS_PER_SC + p * R) * N_VID
            for lbuf, ibuf in zip(lbufs, ibufs):
                mk_idx(lbuf, ibuf, lo)
                pltpu.sync_copy(ones, acc.at[ibuf], add=True)
            plsc.subcore_barrier()

            off = s * SHARE
            pltpu.sync_copy(acc.at[pl.ds(off, SHARE)],
                            out.at[pl.ds(lo + off, SHARE)])
            plsc.subcore_barrier()

            for ibuf in ibufs:
                pltpu.sync_copy(z, acc.at[ibuf])
            plsc.subcore_barrier()
            return 0

        lax.fori_loop(0, NPASS, do_pass, 0)

    phase([lpos, lneg], [idxp, idxn], out_all)
    phase([lpos], [idxp], out_pos)


BT = 200
BV = 2048


def _tc_body(a_ref, b_ref, cnt_ref, o_ref):
    s = lax.dot_general(
        a_ref[...], b_ref[...],
        dimension_numbers=(((1,), (1,)), ((), ())),
        preferred_element_type=jnp.float32,
    )
    o_ref[...] = s * cnt_ref[...]


def _tc_score(h_tag, h_vid, counts):
    return pl.pallas_call(
        _tc_body,
        grid=(N_TAG // BT, N_VID // BV),
        in_specs=[
            pl.BlockSpec((BT, D), lambda i, j: (i, 0)),
            pl.BlockSpec((BV, D), lambda i, j: (j, 0)),
            pl.BlockSpec((BT, BV), lambda i, j: (i, j)),
        ],
        out_specs=pl.BlockSpec((BT, BV), lambda i, j: (i, j)),
        out_shape=jax.ShapeDtypeStruct((N_TAG, N_VID), jnp.float32),
    )(h_tag, h_vid, counts)


def kernel(h_tag, h_vid, pos_tag_idx, pos_vid_idx, neg_tag_idx, neg_vid_idx):
    counts_all, counts_pos = _sc_counts(
        pos_tag_idx, pos_vid_idx, neg_tag_idx, neg_vid_idx
    )
    counts_all = counts_all.reshape(N_TAG, N_VID)
    labels = counts_pos.reshape(N_TAG, N_VID)
    cls_score = _tc_score(h_tag, h_vid, counts_all)
    return (cls_score, labels)
